# baseline (speedup 1.0000x reference)
import functools

import jax
import jax.numpy as jnp
from jax import lax
from jax.experimental import pallas as pl
from jax.experimental.pallas import tpu as pltpu
from jax.experimental.pallas import tpu_sc as plsc

_NC = 2
_NS = 16
_NW = _NC * _NS

_BATCH = 16384
_DIM = 32
_WORDS = _DIM // 2
_ROWS = 1000000
_B_PER_W = _BATCH // _NW
_NSTREAM = 8
_SLEN = _B_PER_W // _NSTREAM


def _gather_body(uidx_hbm, iidx_hbm, utab_hbm, itab_hbm, ue_hbm, ie_hbm,
                 uidx_v, urows_v, iidx_v, irows_v, sem_u, sem_i):
    wid = lax.axis_index("s") * _NC + lax.axis_index("c")
    base = wid * _B_PER_W
    pltpu.sync_copy(uidx_hbm.at[pl.ds(base, _B_PER_W)], uidx_v)
    pltpu.sync_copy(iidx_hbm.at[pl.ds(base, _B_PER_W)], iidx_v)
    copies = []
    for k in range(_NSTREAM):
        s = pl.ds(k * _SLEN, _SLEN)
        copies.append(pltpu.async_copy(
            utab_hbm.at[uidx_v.at[s]], urows_v.at[s], sem_u))
        copies.append(pltpu.async_copy(
            itab_hbm.at[iidx_v.at[s]], irows_v.at[s], sem_i))
    for c in copies:
        c.wait()
    pltpu.sync_copy(urows_v, ue_hbm.at[pl.ds(base, _B_PER_W)])
    pltpu.sync_copy(irows_v, ie_hbm.at[pl.ds(base, _B_PER_W)])


_gather = pl.kernel(
    _gather_body,
    out_type=(
        jax.ShapeDtypeStruct((_BATCH, _WORDS), jnp.int32),
        jax.ShapeDtypeStruct((_BATCH, _WORDS), jnp.int32),
    ),
    mesh=plsc.VectorSubcoreMesh(
        core_axis_name="c", subcore_axis_name="s",
        num_cores=_NC, num_subcores=_NS),
    scratch_types=(
        pltpu.VMEM((_B_PER_W,), jnp.int32),
        pltpu.VMEM((_B_PER_W, _WORDS), jnp.int32),
        pltpu.VMEM((_B_PER_W,), jnp.int32),
        pltpu.VMEM((_B_PER_W, _WORDS), jnp.int32),
        pltpu.SemaphoreType.DMA,
        pltpu.SemaphoreType.DMA,
    ),
    compiler_params=pltpu.CompilerParams(use_tc_tiling_on_sc=False),
)

_BB = 1024


def _mlp_body(ue_ref, ie_ref, w1u_ref, w1i_ref, b1_ref, w2_ref, b2_ref,
              w3t_ref, b3_ref, out_ref):
    h = jnp.dot(ue_ref[...], w1u_ref[...], preferred_element_type=jnp.float32)
    h = h + jnp.dot(ie_ref[...], w1i_ref[...],
                    preferred_element_type=jnp.float32)
    h = jnp.maximum(h + b1_ref[...], 0.0)
    h = jnp.maximum(
        jnp.dot(h, w2_ref[...], preferred_element_type=jnp.float32)
        + b2_ref[...], 0.0)
    out_ref[...] = jnp.sum(h * w3t_ref[...], axis=1) + b3_ref[0, 0]


def _mlp(ue, ie, w1u, w1i, b1, w2, b2, w3t, b3):
    grid = _BATCH // _BB
    full = lambda s: pl.BlockSpec(s, lambda i: (0,) * len(s))
    return pl.pallas_call(
        _mlp_body,
        grid=(grid,),
        in_specs=[
            pl.BlockSpec((_BB, _DIM), lambda i: (i, 0)),
            pl.BlockSpec((_BB, _DIM), lambda i: (i, 0)),
            full((_DIM, 128)),
            full((_DIM, 128)),
            full((1, 128)),
            full((128, 64)),
            full((1, 64)),
            full((1, 64)),
            full((1, 1)),
        ],
        out_specs=pl.BlockSpec((_BB,), lambda i: (i,)),
        out_shape=jax.ShapeDtypeStruct((_BATCH,), jnp.float32),
        compiler_params=pltpu.CompilerParams(
            dimension_semantics=("arbitrary",)),
    )(ue, ie, w1u, w1i, b1, w2, b2, w3t, b3)


def _pack_table(tab):
    p = lax.bitcast_convert_type(
        tab.astype(jnp.bfloat16).reshape(_ROWS // 8, 128, 2), jnp.int32)
    return p.reshape(_ROWS, _WORDS)


def _unpack(words):
    return lax.bitcast_convert_type(words, jnp.bfloat16).reshape(
        _BATCH, _DIM)


@jax.jit
def kernel(user_idx, item_idx, user_table, item_table, W1, b1, W2, b2, W3, b3):
    up = _pack_table(user_table)
    ip = _pack_table(item_table)
    ueW, ieW = _gather(user_idx.astype(jnp.int32), item_idx.astype(jnp.int32),
                       up, ip)
    ue = _unpack(ueW)
    ie = _unpack(ieW)
    w1 = W1.astype(jnp.bfloat16)
    return _mlp(ue, ie, w1[:_DIM], w1[_DIM:], b1.reshape(1, 128),
                W2, b2.reshape(1, 64), W3.reshape(1, 64), b3.reshape(1, 1))


# speedup vs baseline: 15.9683x; 15.9683x over previous
import functools

import jax
import jax.numpy as jnp
from jax import lax
from jax.experimental import pallas as pl
from jax.experimental.pallas import tpu as pltpu
from jax.experimental.pallas import tpu_sc as plsc

_NC = 2
_NS = 16
_NW = _NC * _NS

_BATCH = 16384
_DIM = 32
_PACK = 8
_PROWS = 1000000 // _PACK
_PW = _DIM * _PACK
_B_PER_W = _BATCH // _NW
_CH = 128
_HALF = 256


def _gather_body(uj_hbm, ij_hbm, up_hbm, ip_hbm, ue4_hbm, ie4_hbm,
                 uj_v, ij_v, rows_v, sem):
    wid = lax.axis_index("s") * _NC + lax.axis_index("c")
    base = wid * _B_PER_W
    pltpu.sync_copy(uj_hbm.at[pl.ds(base, _B_PER_W)], uj_v)
    pltpu.sync_copy(ij_hbm.at[pl.ds(base, _B_PER_W)], ij_v)
    for jv, ph, oh in ((uj_v, up_hbm, ue4_hbm), (ij_v, ip_hbm, ie4_hbm)):
        for h in range(_B_PER_W // _HALF):
            cs = [pltpu.async_copy(
                      ph.at[jv.at[pl.ds(h * _HALF + q * _CH, _CH)]],
                      rows_v.at[pl.ds(q * _CH, _CH)], sem)
                  for q in range(_HALF // _CH)]
            for c in cs:
                c.wait()
            pltpu.sync_copy(rows_v, oh.at[pl.ds(base + h * _HALF, _HALF)])


_gather = pl.kernel(
    _gather_body,
    out_type=(
        jax.ShapeDtypeStruct((_BATCH, _PW), jnp.float32),
        jax.ShapeDtypeStruct((_BATCH, _PW), jnp.float32),
    ),
    mesh=plsc.VectorSubcoreMesh(
        core_axis_name="c", subcore_axis_name="s",
        num_cores=_NC, num_subcores=_NS),
    scratch_types=(
        pltpu.VMEM((_B_PER_W,), jnp.int32),
        pltpu.VMEM((_B_PER_W,), jnp.int32),
        pltpu.VMEM((_HALF, _PW), jnp.float32),
        pltpu.SemaphoreType.DMA,
    ),
)

_BB = 1024


def _mlp_body(ue4_ref, ie4_ref, us_ref, is_ref, w1u8_ref, w1i8_ref, b1_ref,
              w2_ref, b2_ref, w3t_ref, b3_ref, out_ref):
    grp = lax.broadcasted_iota(jnp.int32, (1, _PW), 1) // _DIM
    mu = (grp == us_ref[...]).astype(jnp.float32)
    mi = (grp == is_ref[...]).astype(jnp.float32)
    h = jnp.dot(ue4_ref[...] * mu, w1u8_ref[...],
                preferred_element_type=jnp.float32)
    h = h + jnp.dot(ie4_ref[...] * mi, w1i8_ref[...],
                    preferred_element_type=jnp.float32)
    h = jnp.maximum(h + b1_ref[...], 0.0)
    h = jnp.maximum(
        jnp.dot(h, w2_ref[...], preferred_element_type=jnp.float32)
        + b2_ref[...], 0.0)
    out_ref[...] = jnp.sum(h * w3t_ref[...], axis=1) + b3_ref[0, 0]


def _mlp(ue4, ie4, us, isx, w1u8, w1i8, b1, w2, b2, w3t, b3):
    grid = _BATCH // _BB
    full = lambda s: pl.BlockSpec(s, lambda i: (0,) * len(s))
    return pl.pallas_call(
        _mlp_body,
        grid=(grid,),
        in_specs=[
            pl.BlockSpec((_BB, _PW), lambda i: (i, 0)),
            pl.BlockSpec((_BB, _PW), lambda i: (i, 0)),
            pl.BlockSpec((_BB, 1), lambda i: (i, 0)),
            pl.BlockSpec((_BB, 1), lambda i: (i, 0)),
            full((_PW, 128)),
            full((_PW, 128)),
            full((1, 128)),
            full((128, 64)),
            full((1, 64)),
            full((1, 64)),
            full((1, 1)),
        ],
        out_specs=pl.BlockSpec((_BB,), lambda i: (i,)),
        out_shape=jax.ShapeDtypeStruct((_BATCH,), jnp.float32),
        compiler_params=pltpu.CompilerParams(
            dimension_semantics=("arbitrary",)),
    )(ue4, ie4, us, isx, w1u8, w1i8, b1, w2, b2, w3t, b3)


@jax.jit
def kernel(user_idx, item_idx, user_table, item_table, W1, b1, W2, b2, W3, b3):
    ui = user_idx.astype(jnp.int32)
    ii = item_idx.astype(jnp.int32)
    up = user_table.reshape(_PROWS, _PW)
    ip = item_table.reshape(_PROWS, _PW)
    ue4, ie4 = _gather(ui // _PACK, ii // _PACK, up, ip)
    w1u8 = jnp.tile(W1[:_DIM], (_PACK, 1))
    w1i8 = jnp.tile(W1[_DIM:], (_PACK, 1))
    return _mlp(ue4, ie4, (ui % _PACK).reshape(_BATCH, 1),
                (ii % _PACK).reshape(_BATCH, 1), w1u8, w1i8,
                b1.reshape(1, 128), W2, b2.reshape(1, 64),
                W3.reshape(1, 64), b3.reshape(1, 1))
